# baseline (speedup 1.0000x reference)
import functools

import jax
import jax.numpy as jnp
from jax import lax
from jax.experimental import pallas as pl
from jax.experimental.pallas import tpu as pltpu
from jax.experimental.pallas import tpu_sc as plsc

_B = 8
_N = 2048
_EMB = 128
_MAXD = 25

_ROWS = _B * _N
_CR = 64
_NSTEP = _ROWS // _CR
_NBUF = 8

_INFO = plsc.get_sparse_core_info()
_NC = _INFO.num_cores
_NS = _INFO.num_subcores
_NW = _NC * _NS
_RPW = _ROWS // _NW
_TBL = (_MAXD + 1) * _EMB
_HROWS = _RPW // 2


def _deg_kernel(adj_hbm, idx_ref, buf, sems):
    def chunk_copy(t, slot):
        return pltpu.make_async_copy(
            adj_hbm.at[pl.ds(t * _CR, _CR), :], buf.at[slot], sems.at[slot]
        )

    for s in range(_NBUF):
        chunk_copy(s, s).start()

    def body(g, _):
        t0 = g * _NBUF
        for s in range(_NBUF):
            t = t0 + s
            chunk_copy(t, s).wait()
            deg = jnp.sum(buf[s], axis=1)
            idx = jnp.minimum(jnp.round(deg), float(_MAXD))
            idx = jnp.maximum(idx, 0.0).astype(jnp.int32)
            idx_ref[pl.ds(t, 1), :] = idx.reshape(1, _CR)

            @pl.when(t + _NBUF < _NSTEP)
            def _():
                chunk_copy(t + _NBUF, s).start()

        return 0

    lax.fori_loop(0, _NSTEP // _NBUF, body, 0)


_deg_call = pl.pallas_call(
    _deg_kernel,
    in_specs=[pl.BlockSpec(memory_space=pltpu.MemorySpace.HBM)],
    out_specs=pl.BlockSpec(memory_space=pltpu.MemorySpace.VMEM),
    out_shape=jax.ShapeDtypeStruct((_NSTEP, _CR), jnp.int32),
    scratch_shapes=[
        pltpu.VMEM((_NBUF, _CR, _N), jnp.float32),
        pltpu.SemaphoreType.DMA((_NBUF,)),
    ],
)


@functools.partial(
    pl.kernel,
    out_type=jax.ShapeDtypeStruct((_ROWS * _EMB,), jnp.float32),
    mesh=plsc.VectorSubcoreMesh(core_axis_name="c", subcore_axis_name="s"),
    compiler_params=pltpu.CompilerParams(needs_layout_passes=False),
    scratch_types=[
        pltpu.VMEM((_RPW,), jnp.int32),
        pltpu.VMEM((_TBL,), jnp.float32),
        pltpu.VMEM((_HROWS * _EMB,), jnp.float32),
    ],
)
def _gather_kernel(idx_hbm, table_hbm, out_hbm, idxs_v, table_v, rows_v):
    wid = lax.axis_index("s") * _NC + lax.axis_index("c")
    row0 = wid * _RPW
    pltpu.sync_copy(table_hbm, table_v)
    pltpu.sync_copy(idx_hbm.at[pl.ds(row0, _RPW)], idxs_v)

    for h in range(2):
        def group_gather(g, _, _h=h):
            idx16 = idxs_v[pl.ds(_h * _HROWS + g * 16, 16)] * _EMB
            for rr in range(16):
                t = idx16[rr]
                d = (g * 16 + rr) * _EMB
                for cg in range(_EMB // 16):
                    rows_v[pl.ds(d + cg * 16, 16)] = table_v[pl.ds(t + cg * 16, 16)]
            return 0

        lax.fori_loop(0, _HROWS // 16, group_gather, 0)
        pltpu.sync_copy(
            rows_v, out_hbm.at[pl.ds((row0 + h * _HROWS) * _EMB, _HROWS * _EMB)]
        )


def kernel(data, adj, dense, emb_weight):
    idx = _deg_call(adj.reshape(_ROWS, _N))
    out = _gather_kernel(idx.reshape(_ROWS), emb_weight.reshape(_TBL))
    return out.reshape(_B, _N, _EMB)


# speedup vs baseline: 1.0809x; 1.0809x over previous
import functools

import jax
import jax.numpy as jnp
from jax import lax
from jax.experimental import pallas as pl
from jax.experimental.pallas import tpu as pltpu
from jax.experimental.pallas import tpu_sc as plsc

_B = 8
_N = 2048
_EMB = 128
_MAXD = 25

_ROWS = _B * _N
_CR = 128
_NSTEP = _ROWS // _CR
_NBUF = 8

_INFO = plsc.get_sparse_core_info()
_NC = _INFO.num_cores
_NS = _INFO.num_subcores
_NW = _NC * _NS
_RPW = _ROWS // _NW
_TBL = (_MAXD + 1) * _EMB
_HROWS = _RPW // 2


def _deg_kernel(adj_hbm, idx_ref, buf, sems):
    def chunk_copy(t, slot):
        return pltpu.make_async_copy(
            adj_hbm.at[pl.ds(t * _CR, _CR), :], buf.at[slot], sems.at[slot]
        )

    for s in range(_NBUF):
        chunk_copy(s, s).start()

    def body(g, _):
        t0 = g * _NBUF
        for s in range(_NBUF):
            t = t0 + s
            chunk_copy(t, s).wait()
            deg = jnp.sum(buf[s], axis=1)
            idx = jnp.minimum(jnp.round(deg), float(_MAXD))
            idx = jnp.maximum(idx, 0.0).astype(jnp.int32)
            idx_ref[pl.ds(t, 1), :] = idx.reshape(1, _CR)

            @pl.when(t + _NBUF < _NSTEP)
            def _():
                chunk_copy(t + _NBUF, s).start()

        return 0

    lax.fori_loop(0, _NSTEP // _NBUF, body, 0)


_deg_call = pl.pallas_call(
    _deg_kernel,
    in_specs=[pl.BlockSpec(memory_space=pltpu.MemorySpace.HBM)],
    out_specs=pl.BlockSpec(memory_space=pltpu.MemorySpace.VMEM),
    out_shape=jax.ShapeDtypeStruct((_NSTEP, _CR), jnp.int32),
    scratch_shapes=[
        pltpu.VMEM((_NBUF, _CR, _N), jnp.float32),
        pltpu.SemaphoreType.DMA((_NBUF,)),
    ],
)


@functools.partial(
    pl.kernel,
    out_type=jax.ShapeDtypeStruct((_ROWS * _EMB,), jnp.float32),
    mesh=plsc.VectorSubcoreMesh(core_axis_name="c", subcore_axis_name="s"),
    compiler_params=pltpu.CompilerParams(needs_layout_passes=False),
    scratch_types=[
        pltpu.VMEM((_RPW,), jnp.int32),
        pltpu.VMEM((_TBL,), jnp.float32),
        pltpu.VMEM((_HROWS * _EMB,), jnp.float32),
    ],
)
def _gather_kernel(idx_hbm, table_hbm, out_hbm, idxs_v, table_v, rows_v):
    wid = lax.axis_index("s") * _NC + lax.axis_index("c")
    row0 = wid * _RPW
    pltpu.sync_copy(table_hbm, table_v)
    pltpu.sync_copy(idx_hbm.at[pl.ds(row0, _RPW)], idxs_v)

    for h in range(2):
        def group_gather(g, _, _h=h):
            idx16 = idxs_v[pl.ds(_h * _HROWS + g * 16, 16)] * _EMB
            for rr in range(16):
                t = idx16[rr]
                d = (g * 16 + rr) * _EMB
                for cg in range(_EMB // 16):
                    rows_v[pl.ds(d + cg * 16, 16)] = table_v[pl.ds(t + cg * 16, 16)]
            return 0

        lax.fori_loop(0, _HROWS // 16, group_gather, 0)
        pltpu.sync_copy(
            rows_v, out_hbm.at[pl.ds((row0 + h * _HROWS) * _EMB, _HROWS * _EMB)]
        )


def kernel(data, adj, dense, emb_weight):
    idx = _deg_call(adj.reshape(_ROWS, _N))
    out = _gather_kernel(idx.reshape(_ROWS), emb_weight.reshape(_TBL))
    return out.reshape(_B, _N, _EMB)
